# baseline (speedup 1.0000x reference)
import functools

import jax
import jax.numpy as jnp
from jax import lax
from jax.experimental import pallas as pl
from jax.experimental.pallas import tpu as pltpu
from jax.experimental.pallas import tpu_sc as plsc

F32 = jnp.float32
NC = 2
NS = 16
NW = NC * NS
C = 128
G = 8


def _sc_mesh():
    return plsc.VectorSubcoreMesh(
        core_axis_name="c", subcore_axis_name="s",
        num_cores=NC, num_subcores=NS)


def _make_deg(n, nch):
    rpt = n // NS

    @functools.partial(
        pl.kernel,
        out_type=jax.ShapeDtypeStruct((NC, n), F32),
        mesh=_sc_mesh(),
        scratch_types=[
            pltpu.VMEM((nch, C), jnp.int32),
            pltpu.VMEM((C,), F32),
            pltpu.VMEM_SHARED((n,), F32),
        ],
    )
    def deg_kernel(dst_hbm, ones_hbm, zero_hbm, out_hbm, dstv, onesv, acc):
        cid = lax.axis_index("c")
        sid = lax.axis_index("s")
        wid = sid * NC + cid
        pltpu.sync_copy(dst_hbm.at[wid], dstv)
        pltpu.sync_copy(ones_hbm, onesv)
        pltpu.sync_copy(zero_hbm, acc.at[pl.ds(sid * rpt, rpt)])
        plsc.subcore_barrier()

        def step(j, carry):
            pltpu.sync_copy(onesv, acc.at[dstv.at[j]], add=True)
            return carry
        lax.fori_loop(0, nch, step, 0)

        plsc.subcore_barrier()
        pltpu.sync_copy(acc.at[pl.ds(sid * rpt, rpt)],
                        out_hbm.at[cid].at[pl.ds(sid * rpt, rpt)])

    return deg_kernel


def _make_msg(n, d, nch):
    rpt = n // NS
    npair = nch // (2 * G)

    @functools.partial(
        pl.kernel,
        out_type=jax.ShapeDtypeStruct((NC, n, d), F32),
        mesh=_sc_mesh(),
        scratch_types=[
            pltpu.VMEM((2 * G, C), jnp.int32),
            pltpu.VMEM((2 * G, C), jnp.int32),
            pltpu.VMEM((C, d), F32),
            pltpu.VMEM((C, d), F32),
            pltpu.VMEM_SHARED((n, d), F32),
            pltpu.SemaphoreType.DMA,
            pltpu.SemaphoreType.DMA,
            pltpu.SemaphoreType.DMA,
            pltpu.SemaphoreType.DMA,
            pltpu.SemaphoreType.DMA,
            pltpu.SemaphoreType.DMA,
        ],
    )
    def msg_kernel(hs_hbm, src_hbm, dst_hbm, zero_hbm, out_hbm,
                   sidx, didx, r0, r1, acc,
                   siA, siB, diA, diB, sr0, sr1):
        cid = lax.axis_index("c")
        sid = lax.axis_index("s")
        wid = sid * NC + cid
        my_src = src_hbm.at[wid]
        my_dst = dst_hbm.at[wid]
        rbuf = (r0, r1)
        rsem = (sr0, sr1)

        def idx_copies(g, half):
            off = half * G
            s = pltpu.make_async_copy(
                my_src.at[pl.ds(g * G, G)], sidx.at[pl.ds(off, G)],
                siA if half == 0 else siB)
            t = pltpu.make_async_copy(
                my_dst.at[pl.ds(g * G, G)], didx.at[pl.ds(off, G)],
                diA if half == 0 else diB)
            return s, t

        def gather(row, pr):
            return pltpu.make_async_copy(
                hs_hbm.at[sidx.at[row]], rbuf[pr], rsem[pr])

        for cp in idx_copies(0, 0):
            cp.start()
        pltpu.sync_copy(zero_hbm, acc.at[pl.ds(sid * rpt, rpt)])
        for cp in idx_copies(0, 0):
            cp.wait()
        if npair * 2 > 1:
            for cp in idx_copies(1, 1):
                cp.start()
        gather(0, 0).start()
        plsc.subcore_barrier()

        def pair(p, carry):
            for half in (0, 1):
                base = (2 * p + half) * G
                for k in range(G):
                    row = half * G + k
                    pr = k % 2
                    gather(row, pr).wait()
                    if k + 1 < G:
                        gather(row + 1, 1 - pr).start()
                    elif half == 0:
                        for cp in idx_copies(2 * p + 1, 1):
                            cp.wait()
                        gather(G, 1 - pr).start()
                    else:
                        @pl.when(p + 1 < npair)
                        def _():
                            for cp in idx_copies(2 * p + 2, 0):
                                cp.wait()
                            gather(0, 1 - pr).start()
                    pltpu.sync_copy(rbuf[pr], acc.at[didx.at[row]], add=True)
                @pl.when(p + 1 < npair)
                def _():
                    for cp in idx_copies(2 * p + 2 + half, half):
                        cp.start()
            return carry
        lax.fori_loop(0, npair, pair, 0)

        plsc.subcore_barrier()
        pltpu.sync_copy(acc.at[pl.ds(sid * rpt, rpt)],
                        out_hbm.at[cid].at[pl.ds(sid * rpt, rpt)])

    return msg_kernel


def _dinv_of(deg_blk):
    return lax.rsqrt(deg_blk[0] + deg_blk[1] + 1.0)


def _tc_first(degp, x, w):
    n, d = x.shape
    blk = n // 10

    def body(deg_ref, x_ref, w_ref, o_ref):
        dinv = _dinv_of(deg_ref[...])
        o_ref[...] = jnp.dot(x_ref[...], w_ref[...],
                             preferred_element_type=F32) * dinv

    return pl.pallas_call(
        body,
        grid=(n // blk,),
        in_specs=[
            pl.BlockSpec((NC, blk, 1), lambda i: (0, i, 0)),
            pl.BlockSpec((blk, d), lambda i: (i, 0)),
            pl.BlockSpec((d, d), lambda i: (0, 0)),
        ],
        out_specs=pl.BlockSpec((blk, d), lambda i: (i, 0)),
        out_shape=jax.ShapeDtypeStruct((n, d), F32),
    )(degp, x, w)


def _tc_mid(degp, p, hs, b, w):
    n, d = hs.shape
    blk = n // 10

    def body(deg_ref, p_ref, hs_ref, b_ref, w_ref, o_ref):
        dinv = _dinv_of(deg_ref[...])
        pp = p_ref[...]
        o = jnp.maximum((pp[0] + pp[1] + hs_ref[...]) * dinv + b_ref[...], 0.0)
        o_ref[...] = jnp.dot(o, w_ref[...], preferred_element_type=F32) * dinv

    return pl.pallas_call(
        body,
        grid=(n // blk,),
        in_specs=[
            pl.BlockSpec((NC, blk, 1), lambda i: (0, i, 0)),
            pl.BlockSpec((NC, blk, d), lambda i: (0, i, 0)),
            pl.BlockSpec((blk, d), lambda i: (i, 0)),
            pl.BlockSpec((1, d), lambda i: (0, 0)),
            pl.BlockSpec((d, d), lambda i: (0, 0)),
        ],
        out_specs=pl.BlockSpec((blk, d), lambda i: (i, 0)),
        out_shape=jax.ShapeDtypeStruct((n, d), F32),
    )(degp, p, hs, b, w)


def _tc_final(degp, p, hs, b):
    n, d = hs.shape
    blk = n // 10

    def body(deg_ref, p_ref, hs_ref, b_ref, o_ref):
        dinv = _dinv_of(deg_ref[...])
        pp = p_ref[...]
        o_ref[...] = jnp.maximum(
            (pp[0] + pp[1] + hs_ref[...]) * dinv + b_ref[...], 0.0)

    return pl.pallas_call(
        body,
        grid=(n // blk,),
        in_specs=[
            pl.BlockSpec((NC, blk, 1), lambda i: (0, i, 0)),
            pl.BlockSpec((NC, blk, d), lambda i: (0, i, 0)),
            pl.BlockSpec((blk, d), lambda i: (i, 0)),
            pl.BlockSpec((1, d), lambda i: (0, 0)),
        ],
        out_specs=pl.BlockSpec((blk, d), lambda i: (i, 0)),
        out_shape=jax.ShapeDtypeStruct((n, d), F32),
    )(degp, p, hs, b)


def kernel(x, edge_index, W1, b1, W2, b2, W3, b3):
    n, d = x.shape
    e = edge_index.shape[1]
    nch = (e + NW * C - 1) // (NW * C)
    nch = ((nch + 2 * G - 1) // (2 * G)) * (2 * G)
    ep = NW * nch * C
    n_deg = ((n + NS * 128 - 1) // (NS * 128)) * (NS * 128)
    n_acc = n_deg

    trash = n + jnp.arange(ep - e, dtype=jnp.int32) % (n_acc - n)
    src = jnp.pad(edge_index[0].astype(jnp.int32), (0, ep - e)
                  ).reshape(NW, nch, C)
    dst = jnp.concatenate([edge_index[1].astype(jnp.int32), trash]
                          ).reshape(NW, nch, C)
    ones_r = jnp.ones((C,), F32)
    zdeg = jnp.zeros((n_deg // NS,), F32)
    zmsg = jnp.zeros((n_acc // NS, d), F32)

    deg_fn = _make_deg(n_deg, nch)
    msg_fn = _make_msg(n_acc, d, nch)

    degp = deg_fn(dst, ones_r, zdeg).reshape(NC, n_deg, 1)
    hs1 = _tc_first(degp, x, W1)
    p1 = msg_fn(hs1, src, dst, zmsg)
    hs2 = _tc_mid(degp, p1, hs1, b1.reshape(1, -1), W2)
    p2 = msg_fn(hs2, src, dst, zmsg)
    hs3 = _tc_mid(degp, p2, hs2, b2.reshape(1, -1), W3)
    p3 = msg_fn(hs3, src, dst, zmsg)
    return _tc_final(degp, p3, hs3, b3.reshape(1, -1))


# speedup vs baseline: 1.3742x; 1.3742x over previous
import functools

import jax
import jax.numpy as jnp
from jax import lax
from jax.experimental import pallas as pl
from jax.experimental.pallas import tpu as pltpu
from jax.experimental.pallas import tpu_sc as plsc

F32 = jnp.float32
NC = 2
NS = 16
NW = NC * NS
C = 128


def _sc_mesh():
    return plsc.VectorSubcoreMesh(
        core_axis_name="c", subcore_axis_name="s",
        num_cores=NC, num_subcores=NS)


def _make_deg(n, nch):
    rpt = n // NS

    @functools.partial(
        pl.kernel,
        out_type=jax.ShapeDtypeStruct((NC, n), F32),
        mesh=_sc_mesh(),
        scratch_types=[
            pltpu.VMEM((nch, C), jnp.int32),
            pltpu.VMEM((C,), F32),
            pltpu.VMEM_SHARED((n,), F32),
        ],
    )
    def deg_kernel(dst_hbm, ones_hbm, zero_hbm, out_hbm, dstv, onesv, acc):
        cid = lax.axis_index("c")
        sid = lax.axis_index("s")
        wid = sid * NC + cid
        pltpu.sync_copy(dst_hbm.at[wid], dstv)
        pltpu.sync_copy(ones_hbm, onesv)
        pltpu.sync_copy(zero_hbm, acc.at[pl.ds(sid * rpt, rpt)])
        plsc.subcore_barrier()

        def step(j, carry):
            pltpu.sync_copy(onesv, acc.at[dstv.at[j]], add=True)
            return carry
        lax.fori_loop(0, nch, step, 0)

        plsc.subcore_barrier()
        pltpu.sync_copy(acc.at[pl.ds(sid * rpt, rpt)],
                        out_hbm.at[cid].at[pl.ds(sid * rpt, rpt)])

    return deg_kernel


def _make_msg(n, d, nch):
    rpt = n // NS

    @functools.partial(
        pl.kernel,
        out_type=jax.ShapeDtypeStruct((NC, n, d), F32),
        mesh=_sc_mesh(),
        scratch_types=[
            pltpu.VMEM((nch, C), jnp.int32),
            pltpu.VMEM((nch, C), jnp.int32),
            pltpu.VMEM((C, d), F32),
            pltpu.VMEM_SHARED((n, d), F32),
            pltpu.SemaphoreType.DMA,
        ],
    )
    def msg_kernel(hs_hbm, src_hbm, dst_hbm, zero_hbm, out_hbm,
                   srcv, dstv, rowsv, acc, sem):
        cid = lax.axis_index("c")
        sid = lax.axis_index("s")
        wid = sid * NC + cid
        pltpu.sync_copy(src_hbm.at[wid], srcv)
        pltpu.async_copy(hs_hbm.at[srcv.at[0]], rowsv, sem)
        pltpu.sync_copy(dst_hbm.at[wid], dstv)
        pltpu.sync_copy(zero_hbm, acc.at[pl.ds(sid * rpt, rpt)])
        plsc.subcore_barrier()

        def step(j, carry):
            pltpu.make_async_copy(hs_hbm.at[srcv.at[j]], rowsv, sem).wait()
            pltpu.sync_copy(rowsv, acc.at[dstv.at[j]], add=True)

            @pl.when(j + 1 < nch)
            def _():
                pltpu.async_copy(hs_hbm.at[srcv.at[j + 1]], rowsv, sem)
            return carry
        lax.fori_loop(0, nch, step, 0)

        plsc.subcore_barrier()
        pltpu.sync_copy(acc.at[pl.ds(sid * rpt, rpt)],
                        out_hbm.at[cid].at[pl.ds(sid * rpt, rpt)])

    return msg_kernel


def _dinv_of(deg_blk):
    return lax.rsqrt(deg_blk[0] + deg_blk[1] + 1.0)


def _tc_first(degp, x, w):
    n, d = x.shape
    blk = n // 10

    def body(deg_ref, x_ref, w_ref, o_ref):
        dinv = _dinv_of(deg_ref[...])
        o_ref[...] = jnp.dot(x_ref[...], w_ref[...],
                             preferred_element_type=F32) * dinv

    return pl.pallas_call(
        body,
        grid=(n // blk,),
        in_specs=[
            pl.BlockSpec((NC, blk, 1), lambda i: (0, i, 0)),
            pl.BlockSpec((blk, d), lambda i: (i, 0)),
            pl.BlockSpec((d, d), lambda i: (0, 0)),
        ],
        out_specs=pl.BlockSpec((blk, d), lambda i: (i, 0)),
        out_shape=jax.ShapeDtypeStruct((n, d), F32),
    )(degp, x, w)


def _tc_mid(degp, p, hs, b, w):
    n, d = hs.shape
    blk = n // 10

    def body(deg_ref, p_ref, hs_ref, b_ref, w_ref, o_ref):
        dinv = _dinv_of(deg_ref[...])
        pp = p_ref[...]
        o = jnp.maximum((pp[0] + pp[1] + hs_ref[...]) * dinv + b_ref[...], 0.0)
        o_ref[...] = jnp.dot(o, w_ref[...], preferred_element_type=F32) * dinv

    return pl.pallas_call(
        body,
        grid=(n // blk,),
        in_specs=[
            pl.BlockSpec((NC, blk, 1), lambda i: (0, i, 0)),
            pl.BlockSpec((NC, blk, d), lambda i: (0, i, 0)),
            pl.BlockSpec((blk, d), lambda i: (i, 0)),
            pl.BlockSpec((1, d), lambda i: (0, 0)),
            pl.BlockSpec((d, d), lambda i: (0, 0)),
        ],
        out_specs=pl.BlockSpec((blk, d), lambda i: (i, 0)),
        out_shape=jax.ShapeDtypeStruct((n, d), F32),
    )(degp, p, hs, b, w)


def _tc_final(degp, p, hs, b):
    n, d = hs.shape
    blk = n // 10

    def body(deg_ref, p_ref, hs_ref, b_ref, o_ref):
        dinv = _dinv_of(deg_ref[...])
        pp = p_ref[...]
        o_ref[...] = jnp.maximum(
            (pp[0] + pp[1] + hs_ref[...]) * dinv + b_ref[...], 0.0)

    return pl.pallas_call(
        body,
        grid=(n // blk,),
        in_specs=[
            pl.BlockSpec((NC, blk, 1), lambda i: (0, i, 0)),
            pl.BlockSpec((NC, blk, d), lambda i: (0, i, 0)),
            pl.BlockSpec((blk, d), lambda i: (i, 0)),
            pl.BlockSpec((1, d), lambda i: (0, 0)),
        ],
        out_specs=pl.BlockSpec((blk, d), lambda i: (i, 0)),
        out_shape=jax.ShapeDtypeStruct((n, d), F32),
    )(degp, p, hs, b)


def kernel(x, edge_index, W1, b1, W2, b2, W3, b3):
    n, d = x.shape
    e = edge_index.shape[1]
    nch = (e + NW * C - 1) // (NW * C)
    ep = NW * nch * C
    n_deg = ((n + NS * 128 - 1) // (NS * 128)) * (NS * 128)
    n_acc = n_deg

    trash = n + jnp.arange(ep - e, dtype=jnp.int32) % (n_acc - n)
    src = jnp.pad(edge_index[0].astype(jnp.int32), (0, ep - e)
                  ).reshape(NW, nch, C)
    dst = jnp.concatenate([edge_index[1].astype(jnp.int32), trash]
                          ).reshape(NW, nch, C)
    ones_r = jnp.ones((C,), F32)
    zdeg = jnp.zeros((n_deg // NS,), F32)
    zmsg = jnp.zeros((n_acc // NS, d), F32)

    deg_fn = _make_deg(n_deg, nch)
    msg_fn = _make_msg(n_acc, d, nch)

    degp = deg_fn(dst, ones_r, zdeg).reshape(NC, n_deg, 1)
    hs1 = _tc_first(degp, x, W1)
    p1 = msg_fn(hs1, src, dst, zmsg)
    hs2 = _tc_mid(degp, p1, hs1, b1.reshape(1, -1), W2)
    p2 = msg_fn(hs2, src, dst, zmsg)
    hs3 = _tc_mid(degp, p2, hs2, b2.reshape(1, -1), W3)
    p3 = msg_fn(hs3, src, dst, zmsg)
    return _tc_final(degp, p3, hs3, b3.reshape(1, -1))


# speedup vs baseline: 2.3536x; 1.7127x over previous
import functools

import jax
import jax.numpy as jnp
from jax import lax
from jax.experimental import pallas as pl
from jax.experimental.pallas import tpu as pltpu
from jax.experimental.pallas import tpu_sc as plsc

F32 = jnp.float32
NC = 2
NS = 16
NW = NC * NS
C = 125


def _sc_mesh():
    return plsc.VectorSubcoreMesh(
        core_axis_name="c", subcore_axis_name="s",
        num_cores=NC, num_subcores=NS)


def _make_deg(n, nch):
    rpt = n // NS

    @functools.partial(
        pl.kernel,
        out_type=jax.ShapeDtypeStruct((NC, n), F32),
        mesh=_sc_mesh(),
        scratch_types=[
            pltpu.VMEM((nch, C), jnp.int32),
            pltpu.VMEM((C,), F32),
            pltpu.VMEM_SHARED((n,), F32),
        ],
    )
    def deg_kernel(dst_hbm, ones_hbm, zero_hbm, out_hbm, dstv, onesv, acc):
        cid = lax.axis_index("c")
        sid = lax.axis_index("s")
        wid = sid * NC + cid
        pltpu.sync_copy(dst_hbm.at[wid], dstv)
        pltpu.sync_copy(ones_hbm, onesv)
        pltpu.sync_copy(zero_hbm, acc.at[pl.ds(sid * rpt, rpt)])
        plsc.subcore_barrier()

        def step(j, carry):
            pltpu.sync_copy(onesv, acc.at[dstv.at[j]], add=True)
            return carry
        lax.fori_loop(0, nch, step, 0)

        plsc.subcore_barrier()
        pltpu.sync_copy(acc.at[pl.ds(sid * rpt, rpt)],
                        out_hbm.at[cid].at[pl.ds(sid * rpt, rpt)])

    return deg_kernel


def _make_msg(n, d, nch):
    rpt = n // NS

    @functools.partial(
        pl.kernel,
        out_type=jax.ShapeDtypeStruct((NC, n, d), F32),
        mesh=_sc_mesh(),
        scratch_types=[
            pltpu.VMEM((nch, C), jnp.int32),
            pltpu.VMEM((nch, C), jnp.int32),
            pltpu.VMEM((C, d), F32),
            pltpu.VMEM_SHARED((n, d), F32),
            pltpu.SemaphoreType.DMA,
        ],
    )
    def msg_kernel(hs_hbm, src_hbm, dst_hbm, zero_hbm, out_hbm,
                   srcv, dstv, rowsv, acc, sem):
        cid = lax.axis_index("c")
        sid = lax.axis_index("s")
        wid = sid * NC + cid
        pltpu.sync_copy(src_hbm.at[wid], srcv)
        pltpu.async_copy(hs_hbm.at[srcv.at[0]], rowsv, sem)
        pltpu.sync_copy(dst_hbm.at[wid], dstv)
        pltpu.sync_copy(zero_hbm, acc.at[pl.ds(sid * rpt, rpt)])
        plsc.subcore_barrier()

        def step(j, carry):
            pltpu.make_async_copy(hs_hbm.at[srcv.at[j]], rowsv, sem).wait()
            pltpu.sync_copy(rowsv, acc.at[dstv.at[j]], add=True)

            @pl.when(j + 1 < nch)
            def _():
                pltpu.async_copy(hs_hbm.at[srcv.at[j + 1]], rowsv, sem)
            return carry
        lax.fori_loop(0, nch, step, 0)

        plsc.subcore_barrier()
        pltpu.sync_copy(acc.at[pl.ds(sid * rpt, rpt)],
                        out_hbm.at[cid].at[pl.ds(sid * rpt, rpt)])

    return msg_kernel


def _dinv_of(deg_blk):
    return lax.rsqrt(deg_blk[0] + deg_blk[1] + 1.0)


def _tc_first(degp, x, w):
    n, d = x.shape
    blk = n // 10

    def body(deg_ref, x_ref, w_ref, o_ref):
        dinv = _dinv_of(deg_ref[...])
        o_ref[...] = jnp.dot(x_ref[...], w_ref[...],
                             preferred_element_type=F32) * dinv

    return pl.pallas_call(
        body,
        grid=(n // blk,),
        in_specs=[
            pl.BlockSpec((NC, blk, 1), lambda i: (0, i, 0)),
            pl.BlockSpec((blk, d), lambda i: (i, 0)),
            pl.BlockSpec((d, d), lambda i: (0, 0)),
        ],
        out_specs=pl.BlockSpec((blk, d), lambda i: (i, 0)),
        out_shape=jax.ShapeDtypeStruct((n, d), F32),
    )(degp, x, w)


def _tc_mid(degp, p, hs, b, w):
    n, d = hs.shape
    blk = n // 10

    def body(deg_ref, p_ref, hs_ref, b_ref, w_ref, o_ref):
        dinv = _dinv_of(deg_ref[...])
        pp = p_ref[...]
        o = jnp.maximum((pp[0] + pp[1] + hs_ref[...]) * dinv + b_ref[...], 0.0)
        o_ref[...] = jnp.dot(o, w_ref[...], preferred_element_type=F32) * dinv

    return pl.pallas_call(
        body,
        grid=(n // blk,),
        in_specs=[
            pl.BlockSpec((NC, blk, 1), lambda i: (0, i, 0)),
            pl.BlockSpec((NC, blk, d), lambda i: (0, i, 0)),
            pl.BlockSpec((blk, d), lambda i: (i, 0)),
            pl.BlockSpec((1, d), lambda i: (0, 0)),
            pl.BlockSpec((d, d), lambda i: (0, 0)),
        ],
        out_specs=pl.BlockSpec((blk, d), lambda i: (i, 0)),
        out_shape=jax.ShapeDtypeStruct((n, d), F32),
    )(degp, p, hs, b, w)


def _tc_final(degp, p, hs, b):
    n, d = hs.shape
    blk = n // 10

    def body(deg_ref, p_ref, hs_ref, b_ref, o_ref):
        dinv = _dinv_of(deg_ref[...])
        pp = p_ref[...]
        o_ref[...] = jnp.maximum(
            (pp[0] + pp[1] + hs_ref[...]) * dinv + b_ref[...], 0.0)

    return pl.pallas_call(
        body,
        grid=(n // blk,),
        in_specs=[
            pl.BlockSpec((NC, blk, 1), lambda i: (0, i, 0)),
            pl.BlockSpec((NC, blk, d), lambda i: (0, i, 0)),
            pl.BlockSpec((blk, d), lambda i: (i, 0)),
            pl.BlockSpec((1, d), lambda i: (0, 0)),
        ],
        out_specs=pl.BlockSpec((blk, d), lambda i: (i, 0)),
        out_shape=jax.ShapeDtypeStruct((n, d), F32),
    )(degp, p, hs, b)


def kernel(x, edge_index, W1, b1, W2, b2, W3, b3):
    n, d = x.shape
    e = edge_index.shape[1]
    nch = (e + NW * C - 1) // (NW * C)
    ep = NW * nch * C
    n_deg = ((n + NS * 128 - 1) // (NS * 128)) * (NS * 128)
    n_acc = n_deg

    trash = n + jnp.arange(ep - e, dtype=jnp.int32) % (n_acc - n)
    src = jnp.pad(edge_index[0].astype(jnp.int32), (0, ep - e)
                  ).reshape(NW, nch, C)
    dst = jnp.concatenate([edge_index[1].astype(jnp.int32), trash]
                          ).reshape(NW, nch, C)
    ones_r = jnp.ones((C,), F32)
    zdeg = jnp.zeros((n_deg // NS,), F32)
    zmsg = jnp.zeros((n_acc // NS, d), F32)

    deg_fn = _make_deg(n_deg, nch)
    msg_fn = _make_msg(n_acc, d, nch)

    degp = deg_fn(dst, ones_r, zdeg).reshape(NC, n_deg, 1)
    hs1 = _tc_first(degp, x, W1)
    p1 = msg_fn(hs1, src, dst, zmsg)
    hs2 = _tc_mid(degp, p1, hs1, b1.reshape(1, -1), W2)
    p2 = msg_fn(hs2, src, dst, zmsg)
    hs3 = _tc_mid(degp, p2, hs2, b2.reshape(1, -1), W3)
    p3 = msg_fn(hs3, src, dst, zmsg)
    return _tc_final(degp, p3, hs3, b3.reshape(1, -1))
